# Initial kernel scaffold; baseline (speedup 1.0000x reference)
import jax
import jax.numpy as jnp
from jax.experimental import pallas as pl


def kernel(positions, scales, rotations, colors, opacities, K, background):
    raise NotImplementedError("write your pallas kernel here")


# speedup vs baseline: 27.2920x; 27.2920x over previous
import jax
import jax.numpy as jnp
from jax.experimental import pallas as pl

H, W = 128, 128
TILE = 16
TH = H // TILE
TW = W // TILE
NT = TH * TW
P = TILE * TILE


def _quat_to_rotmat(q):
    q = q / (jnp.linalg.norm(q, axis=-1, keepdims=True) + 1e-8)
    w, x, y, z = q[:, 0], q[:, 1], q[:, 2], q[:, 3]
    r00 = 1 - 2 * (y * y + z * z); r01 = 2 * (x * y - w * z); r02 = 2 * (x * z + w * y)
    r10 = 2 * (x * y + w * z); r11 = 1 - 2 * (x * x + z * z); r12 = 2 * (y * z - w * x)
    r20 = 2 * (x * z - w * y); r21 = 2 * (y * z + w * x); r22 = 1 - 2 * (x * x + y * y)
    return jnp.stack([jnp.stack([r00, r01, r02], -1),
                      jnp.stack([r10, r11, r12], -1),
                      jnp.stack([r20, r21, r22], -1)], -2)


def _project(positions, scales, rotations, K):
    R = _quat_to_rotmat(rotations)
    M = R * scales[:, None, :]
    Sigma = jnp.einsum('nij,nkj->nik', M, M)
    fx, fy, cx, cy = K[0, 0], K[1, 1], K[0, 2], K[1, 2]
    x, y = positions[:, 0], positions[:, 1]
    z = jnp.clip(positions[:, 2], 1e-3, None)
    means_2d = jnp.stack([fx * x / z + cx, fy * y / z + cy], -1)
    zero = jnp.zeros_like(z)
    J = jnp.stack([jnp.stack([fx / z, zero, -fx * x / (z * z)], -1),
                   jnp.stack([zero, fy / z, -fy * y / (z * z)], -1)], -2)
    cov2d = jnp.einsum('nij,njk,nlk->nil', J, Sigma, J) + 1e-4 * jnp.eye(2, dtype=positions.dtype)
    return means_2d, cov2d, z


def _tile_bounds(means_2d, cov_2d):
    trace = cov_2d[:, 0, 0] + cov_2d[:, 1, 1]
    det = cov_2d[:, 0, 0] * cov_2d[:, 1, 1] - cov_2d[:, 0, 1] * cov_2d[:, 1, 0]
    lam = (trace + jnp.sqrt(jnp.clip(trace ** 2 - 4 * det, 0.0, None))) / 2
    radius = 3.0 * jnp.sqrt(lam + 1e-6)
    tx_min = jnp.clip(jnp.floor((means_2d[:, 0] - radius) / TILE).astype(jnp.int32), 0, TW - 1)
    tx_max = jnp.clip(jnp.ceil((means_2d[:, 0] + radius) / TILE).astype(jnp.int32), 0, TW - 1)
    ty_min = jnp.clip(jnp.floor((means_2d[:, 1] - radius) / TILE).astype(jnp.int32), 0, TH - 1)
    ty_max = jnp.clip(jnp.ceil((means_2d[:, 1] + radius) / TILE).astype(jnp.int32), 0, TH - 1)
    return tx_min, tx_max, ty_min, ty_max


def _raster_kernel(pfa_ref, pfb_ref, o_ref):
    t = pl.program_id(0)
    tyf = (t // TW).astype(jnp.float32)
    txf = (t % TW).astype(jnp.float32)
    idx = jax.lax.broadcasted_iota(jnp.int32, (P, 1), 0)
    iy = (idx // TILE).astype(jnp.float32)
    ix = (idx % TILE).astype(jnp.float32)
    pyf = tyf * TILE + iy
    pxf = txf * TILE + ix
    mx = pfa_ref[0:1, :]
    my = pfa_ref[1:2, :]
    ci00 = pfa_ref[2:3, :]
    cs = pfa_ref[3:4, :]
    ci11 = pfa_ref[4:5, :]
    op = pfa_ref[5:6, :]
    txmin = pfa_ref[6:7, :]
    txmax = pfa_ref[7:8, :]
    tymin = pfa_ref[8:9, :]
    tymax = pfa_ref[9:10, :]
    dx = pxf - mx
    dy = pyf - my
    mahal = dx * dx * ci00 + dx * dy * cs + dy * dy * ci11
    w = jnp.exp(-0.5 * mahal) * op
    tmask = (txmin <= txf) & (txf <= txmax) & (tymin <= tyf) & (tyf <= tymax)
    alpha = jnp.where((w > 0.01) & tmask, w, 0.0)
    lg = jnp.log1p(-alpha)
    cum = lg
    k = 1
    while k < cum.shape[1]:
        z = jnp.zeros((cum.shape[0], k), cum.dtype)
        cum = cum + jnp.concatenate([z, cum[:, :-k]], axis=1)
        k *= 2
    contrib = jnp.exp(cum - lg) * alpha
    out8 = jax.lax.dot_general(pfb_ref[...], contrib,
                               (((1,), (1,)), ((), ())),
                               preferred_element_type=jnp.float32)
    o_ref[0] = out8


def kernel(positions, scales, rotations, colors, opacities, K, background):
    n = positions.shape[0]
    means_2d, cov_2d, depths = _project(positions, scales, rotations, K)
    tx_min, tx_max, ty_min, ty_max = _tile_bounds(means_2d, cov_2d)
    order = jnp.argsort(-depths)
    m = means_2d[order]
    cov = cov_2d[order]
    a, b, c, d = cov[:, 0, 0], cov[:, 0, 1], cov[:, 1, 0], cov[:, 1, 1]
    inv_det = 1.0 / (a * d - b * c + 1e-12)
    ci00 = d * inv_det
    cs = -(b + c) * inv_det
    ci11 = a * inv_det
    col = colors[order]
    op = opacities[order, 0]
    dep = depths[order]
    f32 = lambda v: v.astype(jnp.float32)
    zeros = jnp.zeros((n,), jnp.float32)
    ones = jnp.ones((n,), jnp.float32)
    pfa = jnp.stack([m[:, 0], m[:, 1], ci00, cs, ci11, op,
                     f32(tx_min[order]), f32(tx_max[order]),
                     f32(ty_min[order]), f32(ty_max[order]),
                     zeros, zeros, zeros, zeros, zeros, zeros], axis=0)
    pfb = jnp.stack([col[:, 0], col[:, 1], col[:, 2], dep, ones,
                     zeros, zeros, zeros], axis=0)
    out = pl.pallas_call(
        _raster_kernel,
        grid=(NT,),
        in_specs=[pl.BlockSpec((16, n), lambda t: (0, 0)),
                  pl.BlockSpec((8, n), lambda t: (0, 0))],
        out_specs=pl.BlockSpec((1, 8, P), lambda t: (t, 0, 0)),
        out_shape=jax.ShapeDtypeStruct((NT, 8, P), jnp.float32),
    )(pfa, pfb)
    r = out.reshape(TH, TW, 8, TILE, TILE).transpose(2, 0, 3, 1, 4).reshape(8, H, W)
    img = jnp.clip(background[None, None, :] + r[0:3].transpose(1, 2, 0), 0.0, 1.0)
    return img, r[3], r[4]


# speedup vs baseline: 27.8966x; 1.0222x over previous
import jax
import jax.numpy as jnp
from jax.experimental import pallas as pl

H, W = 128, 128
TILE = 16
TH = H // TILE
TW = W // TILE
NT = TH * TW
P = TILE * TILE


def _quat_to_rotmat(q):
    q = q / (jnp.linalg.norm(q, axis=-1, keepdims=True) + 1e-8)
    w, x, y, z = q[:, 0], q[:, 1], q[:, 2], q[:, 3]
    r00 = 1 - 2 * (y * y + z * z); r01 = 2 * (x * y - w * z); r02 = 2 * (x * z + w * y)
    r10 = 2 * (x * y + w * z); r11 = 1 - 2 * (x * x + z * z); r12 = 2 * (y * z - w * x)
    r20 = 2 * (x * z - w * y); r21 = 2 * (y * z + w * x); r22 = 1 - 2 * (x * x + y * y)
    return jnp.stack([jnp.stack([r00, r01, r02], -1),
                      jnp.stack([r10, r11, r12], -1),
                      jnp.stack([r20, r21, r22], -1)], -2)


def _project(positions, scales, rotations, K):
    R = _quat_to_rotmat(rotations)
    M = R * scales[:, None, :]
    Sigma = jnp.einsum('nij,nkj->nik', M, M)
    fx, fy, cx, cy = K[0, 0], K[1, 1], K[0, 2], K[1, 2]
    x, y = positions[:, 0], positions[:, 1]
    z = jnp.clip(positions[:, 2], 1e-3, None)
    means_2d = jnp.stack([fx * x / z + cx, fy * y / z + cy], -1)
    zero = jnp.zeros_like(z)
    J = jnp.stack([jnp.stack([fx / z, zero, -fx * x / (z * z)], -1),
                   jnp.stack([zero, fy / z, -fy * y / (z * z)], -1)], -2)
    cov2d = jnp.einsum('nij,njk,nlk->nil', J, Sigma, J) + 1e-4 * jnp.eye(2, dtype=positions.dtype)
    return means_2d, cov2d, z


def _tile_bounds(means_2d, cov_2d):
    trace = cov_2d[:, 0, 0] + cov_2d[:, 1, 1]
    det = cov_2d[:, 0, 0] * cov_2d[:, 1, 1] - cov_2d[:, 0, 1] * cov_2d[:, 1, 0]
    lam = (trace + jnp.sqrt(jnp.clip(trace ** 2 - 4 * det, 0.0, None))) / 2
    radius = 3.0 * jnp.sqrt(lam + 1e-6)
    tx_min = jnp.clip(jnp.floor((means_2d[:, 0] - radius) / TILE).astype(jnp.int32), 0, TW - 1)
    tx_max = jnp.clip(jnp.ceil((means_2d[:, 0] + radius) / TILE).astype(jnp.int32), 0, TW - 1)
    ty_min = jnp.clip(jnp.floor((means_2d[:, 1] - radius) / TILE).astype(jnp.int32), 0, TH - 1)
    ty_max = jnp.clip(jnp.ceil((means_2d[:, 1] + radius) / TILE).astype(jnp.int32), 0, TH - 1)
    return tx_min, tx_max, ty_min, ty_max


def _raster_kernel(pfa_ref, pfb_ref, o_ref):
    t = pl.program_id(0)
    tyf = (t // TW).astype(jnp.float32)
    txf = (t % TW).astype(jnp.float32)
    idx = jax.lax.broadcasted_iota(jnp.int32, (P, 1), 0)
    iy = (idx // TILE).astype(jnp.float32)
    ix = (idx % TILE).astype(jnp.float32)
    pyf = tyf * TILE + iy
    pxf = txf * TILE + ix
    mx = pfa_ref[0:1, :]
    my = pfa_ref[1:2, :]
    ci00 = pfa_ref[2:3, :]
    cs = pfa_ref[3:4, :]
    ci11 = pfa_ref[4:5, :]
    op = pfa_ref[5:6, :]
    txmin = pfa_ref[6:7, :]
    txmax = pfa_ref[7:8, :]
    tymin = pfa_ref[8:9, :]
    tymax = pfa_ref[9:10, :]
    dx = pxf - mx
    dy = pyf - my
    mahal = dx * dx * ci00 + dx * dy * cs + dy * dy * ci11
    w = jnp.exp(-0.5 * mahal) * op
    tmask = (txmin <= txf) & (txf <= txmax) & (tymin <= tyf) & (tyf <= tymax)
    alpha = jnp.where((w > 0.01) & tmask, w, 0.0)
    cp = 1.0 - alpha
    k = 1
    while k < cp.shape[1]:
        o = jnp.ones((cp.shape[0], k), cp.dtype)
        cp = cp * jnp.concatenate([o, cp[:, :-k]], axis=1)
        k *= 2
    o1 = jnp.ones((cp.shape[0], 1), cp.dtype)
    texcl = jnp.concatenate([o1, cp[:, :-1]], axis=1)
    contrib = texcl * alpha
    out8 = jax.lax.dot_general(pfb_ref[...], contrib,
                               (((1,), (1,)), ((), ())),
                               preferred_element_type=jnp.float32)
    o_ref[0] = out8


def kernel(positions, scales, rotations, colors, opacities, K, background):
    n = positions.shape[0]
    means_2d, cov_2d, depths = _project(positions, scales, rotations, K)
    tx_min, tx_max, ty_min, ty_max = _tile_bounds(means_2d, cov_2d)
    order = jnp.argsort(-depths)
    m = means_2d[order]
    cov = cov_2d[order]
    a, b, c, d = cov[:, 0, 0], cov[:, 0, 1], cov[:, 1, 0], cov[:, 1, 1]
    inv_det = 1.0 / (a * d - b * c + 1e-12)
    ci00 = d * inv_det
    cs = -(b + c) * inv_det
    ci11 = a * inv_det
    col = colors[order]
    op = opacities[order, 0]
    dep = depths[order]
    f32 = lambda v: v.astype(jnp.float32)
    zeros = jnp.zeros((n,), jnp.float32)
    ones = jnp.ones((n,), jnp.float32)
    pfa = jnp.stack([m[:, 0], m[:, 1], ci00, cs, ci11, op,
                     f32(tx_min[order]), f32(tx_max[order]),
                     f32(ty_min[order]), f32(ty_max[order]),
                     zeros, zeros, zeros, zeros, zeros, zeros], axis=0)
    pfb = jnp.stack([col[:, 0], col[:, 1], col[:, 2], dep, ones,
                     zeros, zeros, zeros], axis=0)
    out = pl.pallas_call(
        _raster_kernel,
        grid=(NT,),
        in_specs=[pl.BlockSpec((16, n), lambda t: (0, 0)),
                  pl.BlockSpec((8, n), lambda t: (0, 0))],
        out_specs=pl.BlockSpec((1, 8, P), lambda t: (t, 0, 0)),
        out_shape=jax.ShapeDtypeStruct((NT, 8, P), jnp.float32),
    )(pfa, pfb)
    r = out.reshape(TH, TW, 8, TILE, TILE).transpose(2, 0, 3, 1, 4).reshape(8, H, W)
    img = jnp.clip(background[None, None, :] + r[0:3].transpose(1, 2, 0), 0.0, 1.0)
    return img, r[3], r[4]


# speedup vs baseline: 33.9792x; 1.2180x over previous
import functools

import jax
import jax.numpy as jnp
from jax import lax
from jax.experimental import pallas as pl
from jax.experimental.pallas import tpu as pltpu
from jax.experimental.pallas import tpu_sc as plsc

H, W = 128, 128
TILE = 16
TH = H // TILE
TW = W // TILE
NT = TH * TW
P = TILE * TILE

_NC, _NS = 2, 16
_NW = _NC * _NS
_TPW = NT // _NW
_G = 128


def _quat_to_rotmat(q):
    q = q / (jnp.linalg.norm(q, axis=-1, keepdims=True) + 1e-8)
    w, x, y, z = q[:, 0], q[:, 1], q[:, 2], q[:, 3]
    r00 = 1 - 2 * (y * y + z * z); r01 = 2 * (x * y - w * z); r02 = 2 * (x * z + w * y)
    r10 = 2 * (x * y + w * z); r11 = 1 - 2 * (x * x + z * z); r12 = 2 * (y * z - w * x)
    r20 = 2 * (x * z - w * y); r21 = 2 * (y * z + w * x); r22 = 1 - 2 * (x * x + y * y)
    return jnp.stack([jnp.stack([r00, r01, r02], -1),
                      jnp.stack([r10, r11, r12], -1),
                      jnp.stack([r20, r21, r22], -1)], -2)


def _project(positions, scales, rotations, K):
    R = _quat_to_rotmat(rotations)
    M = R * scales[:, None, :]
    Sigma = jnp.einsum('nij,nkj->nik', M, M)
    fx, fy, cx, cy = K[0, 0], K[1, 1], K[0, 2], K[1, 2]
    x, y = positions[:, 0], positions[:, 1]
    z = jnp.clip(positions[:, 2], 1e-3, None)
    means_2d = jnp.stack([fx * x / z + cx, fy * y / z + cy], -1)
    zero = jnp.zeros_like(z)
    J = jnp.stack([jnp.stack([fx / z, zero, -fx * x / (z * z)], -1),
                   jnp.stack([zero, fy / z, -fy * y / (z * z)], -1)], -2)
    cov2d = jnp.einsum('nij,njk,nlk->nil', J, Sigma, J) + 1e-4 * jnp.eye(2, dtype=positions.dtype)
    return means_2d, cov2d, z


def _tile_bounds(means_2d, cov_2d):
    trace = cov_2d[:, 0, 0] + cov_2d[:, 1, 1]
    det = cov_2d[:, 0, 0] * cov_2d[:, 1, 1] - cov_2d[:, 0, 1] * cov_2d[:, 1, 0]
    lam = (trace + jnp.sqrt(jnp.clip(trace ** 2 - 4 * det, 0.0, None))) / 2
    radius = 3.0 * jnp.sqrt(lam + 1e-6)
    tx_min = jnp.clip(jnp.floor((means_2d[:, 0] - radius) / TILE).astype(jnp.int32), 0, TW - 1)
    tx_max = jnp.clip(jnp.ceil((means_2d[:, 0] + radius) / TILE).astype(jnp.int32), 0, TW - 1)
    ty_min = jnp.clip(jnp.floor((means_2d[:, 1] - radius) / TILE).astype(jnp.int32), 0, TH - 1)
    ty_max = jnp.clip(jnp.ceil((means_2d[:, 1] + radius) / TILE).astype(jnp.int32), 0, TH - 1)
    return tx_min, tx_max, ty_min, ty_max


def _make_sc_binner(n):
    nchunks = n // 16
    mesh = plsc.VectorSubcoreMesh(core_axis_name="c", subcore_axis_name="s")

    @functools.partial(
        pl.kernel, mesh=mesh,
        out_type=[jax.ShapeDtypeStruct((NT, 16, n), jnp.float32),
                  jax.ShapeDtypeStruct((_NW, 16), jnp.int32)],
        scratch_types=[pltpu.VMEM((16, n), jnp.float32),
                       pltpu.VMEM((n,), jnp.int32),
                       pltpu.VMEM((n,), jnp.int32),
                       pltpu.VMEM((n,), jnp.int32),
                       pltpu.VMEM((n,), jnp.int32),
                       pltpu.VMEM((16, n), jnp.float32),
                       pltpu.VMEM((16,), jnp.int32)],
        compiler_params=pltpu.CompilerParams(needs_layout_passes=False),
    )
    def binner(tablet_hbm, txmin_hbm, txmax_hbm, tymin_hbm, tymax_hbm,
               gath_out, cnt_out,
               table_v, txmin_v, txmax_v, tymin_v, tymax_v, rows_v, cnt_v):
        wid = lax.axis_index("s") * _NC + lax.axis_index("c")
        pltpu.sync_copy(tablet_hbm, table_v)
        pltpu.sync_copy(txmin_hbm, txmin_v)
        pltpu.sync_copy(txmax_hbm, txmax_v)
        pltpu.sync_copy(tymin_hbm, tymin_v)
        pltpu.sync_copy(tymax_hbm, tymax_v)
        iota = lax.broadcasted_iota(jnp.int32, (16,), 0)
        lane15 = jnp.full((16,), 15, jnp.int32)
        counts_vec = jnp.zeros((16,), jnp.int32)
        for k in range(_TPW):
            t = wid * _TPW + k
            ty = t // TW
            tx = t % TW
            base = jnp.zeros((16,), jnp.int32)
            for j in range(nchunks):
                sl = pl.ds(j * 16, 16)
                m = ((txmin_v[sl] <= tx) & (tx <= txmax_v[sl])
                     & (tymin_v[sl] <= ty) & (ty <= tymax_v[sl]))
                cum = plsc.cumsum(jnp.where(m, 1, 0))
                pos = base + cum - 1
                for p in range(16):
                    plsc.store_scatter(
                        rows_v, [jnp.full((16,), p, jnp.int32), pos],
                        table_v[p, sl], mask=m)
                base = base + lax.gather(
                    cum, lane15[:, None],
                    lax.GatherDimensionNumbers(offset_dims=(),
                                               collapsed_slice_dims=(0,),
                                               start_index_map=(0,)),
                    (1,), mode=lax.GatherScatterMode.PROMISE_IN_BOUNDS)
            pltpu.sync_copy(rows_v, gath_out.at[t])
            counts_vec = jnp.where(iota == k, base, counts_vec)
        cnt_v[...] = counts_vec
        pltpu.sync_copy(cnt_v, cnt_out.at[wid])

    return binner


def _raster_kernel(cnt_ref, p_ref, o_ref):
    t = pl.program_id(0)
    tyf = (t // TW).astype(jnp.float32)
    txf = (t % TW).astype(jnp.float32)
    idx = lax.broadcasted_iota(jnp.int32, (P, 1), 0)
    pyf = tyf * TILE + (idx // TILE).astype(jnp.float32)
    pxf = txf * TILE + (idx % TILE).astype(jnp.float32)
    cnt = cnt_ref[t]
    nch = (cnt + _G - 1) // _G

    lane = lax.broadcasted_iota(jnp.int32, (1, _G), 1)

    def body(c, carry):
        acc, tcar = carry
        sl = pl.ds(c * _G, _G)
        inb = (c * _G + lane) < cnt
        mx = p_ref[0, 0:1, sl]
        my = p_ref[0, 1:2, sl]
        ci00 = p_ref[0, 2:3, sl]
        cs = p_ref[0, 3:4, sl]
        ci11 = p_ref[0, 4:5, sl]
        op = p_ref[0, 5:6, sl]
        dx = pxf - mx
        dy = pyf - my
        mahal = dx * dx * ci00 + dx * dy * cs + dy * dy * ci11
        w = jnp.exp(-0.5 * mahal) * op
        alpha = jnp.where((w > 0.01) & inb, w, 0.0)
        cp = 1.0 - alpha
        k = 1
        while k < _G:
            o = jnp.ones((P, k), jnp.float32)
            cp = cp * jnp.concatenate([o, cp[:, :-k]], axis=1)
            k *= 2
        o1 = jnp.ones((P, 1), jnp.float32)
        texcl = jnp.concatenate([o1, cp[:, :-1]], axis=1) * tcar
        contrib = texcl * alpha
        p8 = jnp.where(inb, p_ref[0, 8:16, sl], 0.0)
        acc = acc + lax.dot_general(p8, contrib,
                                    (((1,), (1,)), ((), ())),
                                    preferred_element_type=jnp.float32)
        tcar = tcar * cp[:, -1:]
        return acc, tcar

    acc, _ = lax.fori_loop(
        0, nch, body,
        (jnp.zeros((8, P), jnp.float32), jnp.ones((P, 1), jnp.float32)))
    o_ref[0] = acc


def kernel(positions, scales, rotations, colors, opacities, K, background):
    n = positions.shape[0]
    means_2d, cov_2d, depths = _project(positions, scales, rotations, K)
    tx_min, tx_max, ty_min, ty_max = _tile_bounds(means_2d, cov_2d)
    order = jnp.argsort(-depths)
    m = means_2d[order]
    cov = cov_2d[order]
    a, b, c, d = cov[:, 0, 0], cov[:, 0, 1], cov[:, 1, 0], cov[:, 1, 1]
    inv_det = 1.0 / (a * d - b * c + 1e-12)
    ci00 = d * inv_det
    cs = -(b + c) * inv_det
    ci11 = a * inv_det
    col = colors[order]
    op = opacities[order, 0]
    dep = depths[order]
    zeros = jnp.zeros((n,), jnp.float32)
    ones = jnp.ones((n,), jnp.float32)
    table = jnp.stack([m[:, 0], m[:, 1], ci00, cs, ci11, op, zeros, zeros,
                       col[:, 0], col[:, 1], col[:, 2], dep, ones,
                       zeros, zeros, zeros], axis=0)
    gathered, cnt32 = _make_sc_binner(n)(
        table,
        tx_min[order].astype(jnp.int32), tx_max[order].astype(jnp.int32),
        ty_min[order].astype(jnp.int32), ty_max[order].astype(jnp.int32))
    counts = cnt32[:, :_TPW].reshape(-1)
    grid_spec = pltpu.PrefetchScalarGridSpec(
        num_scalar_prefetch=1,
        grid=(NT,),
        in_specs=[pl.BlockSpec((1, 16, n), lambda t, s: (t, 0, 0))],
        out_specs=pl.BlockSpec((1, 8, P), lambda t, s: (t, 0, 0)),
    )
    out = pl.pallas_call(
        _raster_kernel,
        grid_spec=grid_spec,
        out_shape=jax.ShapeDtypeStruct((NT, 8, P), jnp.float32),
    )(counts, gathered)
    r = out.reshape(TH, TW, 8, TILE, TILE).transpose(2, 0, 3, 1, 4).reshape(8, H, W)
    img = jnp.clip(background[None, None, :] + r[0:3].transpose(1, 2, 0), 0.0, 1.0)
    return img, r[3], r[4]


# speedup vs baseline: 68.7626x; 2.0237x over previous
import functools

import jax
import jax.numpy as jnp
from jax import lax
from jax.experimental import pallas as pl
from jax.experimental.pallas import tpu as pltpu
from jax.experimental.pallas import tpu_sc as plsc

H, W = 128, 128
TILE = 16
TH = H // TILE
TW = W // TILE
NT = TH * TW
P = TILE * TILE

_NC, _NS = 2, 16
_NW = _NC * _NS
_TPW = NT // _NW
_G = 128


def _rolll(x, j):
    return jnp.concatenate([x[:, j:], x[:, :j]], axis=1)


def _rollr(x, j):
    return jnp.concatenate([x[:, -j:], x[:, :-j]], axis=1)


def _prep_kernel(inp_ref, k_ref, table_ref, bounds_ref, order_ref):
    n = inp_ref.shape[1]
    fx = k_ref[0]
    fy = k_ref[1]
    cx = k_ref[2]
    cy = k_ref[3]
    x = inp_ref[0:1, :]
    y = inp_ref[1:2, :]
    z = jnp.maximum(inp_ref[2:3, :], 1e-3)
    s0 = inp_ref[3:4, :]
    s1 = inp_ref[4:5, :]
    s2 = inp_ref[5:6, :]
    qw = inp_ref[6:7, :]
    qx = inp_ref[7:8, :]
    qy = inp_ref[8:9, :]
    qz = inp_ref[9:10, :]
    qn = jnp.sqrt(qw * qw + qx * qx + qy * qy + qz * qz) + 1e-8
    qw = qw / qn; qx = qx / qn; qy = qy / qn; qz = qz / qn
    r00 = 1 - 2 * (qy * qy + qz * qz); r01 = 2 * (qx * qy - qw * qz); r02 = 2 * (qx * qz + qw * qy)
    r10 = 2 * (qx * qy + qw * qz); r11 = 1 - 2 * (qx * qx + qz * qz); r12 = 2 * (qy * qz - qw * qx)
    r20 = 2 * (qx * qz - qw * qy); r21 = 2 * (qy * qz + qw * qx); r22 = 1 - 2 * (qx * qx + qy * qy)
    t0 = s0 * s0; t1 = s1 * s1; t2 = s2 * s2
    s00 = r00 * r00 * t0 + r01 * r01 * t1 + r02 * r02 * t2
    s01 = r00 * r10 * t0 + r01 * r11 * t1 + r02 * r12 * t2
    s02 = r00 * r20 * t0 + r01 * r21 * t1 + r02 * r22 * t2
    s11 = r10 * r10 * t0 + r11 * r11 * t1 + r12 * r12 * t2
    s12 = r10 * r20 * t0 + r11 * r21 * t1 + r12 * r22 * t2
    s22 = r20 * r20 * t0 + r21 * r21 * t1 + r22 * r22 * t2
    mx = fx * x / z + cx
    my = fy * y / z + cy
    a1 = fx / z
    b2 = fy / z
    c1 = -fx * x / (z * z)
    c2 = -fy * y / (z * z)
    cov00 = a1 * a1 * s00 + 2 * a1 * c1 * s02 + c1 * c1 * s22 + 1e-4
    cov01 = a1 * b2 * s01 + a1 * c2 * s02 + c1 * b2 * s12 + c1 * c2 * s22
    cov11 = b2 * b2 * s11 + 2 * b2 * c2 * s12 + c2 * c2 * s22 + 1e-4
    trace = cov00 + cov11
    det = cov00 * cov11 - cov01 * cov01
    lam = (trace + jnp.sqrt(jnp.maximum(trace * trace - 4 * det, 0.0))) * 0.5
    radius = 3.0 * jnp.sqrt(lam + 1e-6)
    ftw = jnp.float32(TW - 1)
    fth = jnp.float32(TH - 1)
    txmin = jnp.clip(jnp.floor((mx - radius) * (1.0 / TILE)), 0.0, ftw).astype(jnp.int32)
    txmax = jnp.clip(jnp.ceil((mx + radius) * (1.0 / TILE)), 0.0, ftw).astype(jnp.int32)
    tymin = jnp.clip(jnp.floor((my - radius) * (1.0 / TILE)), 0.0, fth).astype(jnp.int32)
    tymax = jnp.clip(jnp.ceil((my + radius) * (1.0 / TILE)), 0.0, fth).astype(jnp.int32)
    inv_det = 1.0 / (cov00 * cov11 - cov01 * cov01 + 1e-12)
    ci00 = cov11 * inv_det
    cs = -(cov01 + cov01) * inv_det
    ci11 = cov00 * inv_det
    key = -z
    pay = lax.broadcasted_iota(jnp.int32, (1, n), 1)
    li = lax.broadcasted_iota(jnp.int32, (1, n), 1)
    kk = 2
    while kk <= n:
        jj = kk // 2
        while jj >= 1:
            high = (li & jj) != 0
            desc = (li & kk) != 0
            tk = jnp.where(high, _rollr(key, jj), _rolll(key, jj))
            tp = jnp.where(high, _rollr(pay, jj), _rolll(pay, jj))
            lexgt = (key > tk) | ((key == tk) & (pay > tp))
            take = lexgt ^ high ^ desc
            key = jnp.where(take, tk, key)
            pay = jnp.where(take, tp, pay)
            jj //= 2
        kk *= 2
    order_ref[...] = pay
    table_ref[0:1, :] = mx
    table_ref[1:2, :] = my
    table_ref[2:3, :] = ci00
    table_ref[3:4, :] = cs
    table_ref[4:5, :] = ci11
    table_ref[5:6, :] = inp_ref[13:14, :]
    table_ref[6:9, :] = inp_ref[10:13, :]
    table_ref[9:10, :] = z
    table_ref[10:11, :] = jnp.ones((1, n), jnp.float32)
    table_ref[11:16, :] = jnp.zeros((5, n), jnp.float32)
    bounds_ref[0:1, :] = txmin
    bounds_ref[1:2, :] = txmax
    bounds_ref[2:3, :] = tymin
    bounds_ref[3:4, :] = tymax


def _make_sc_binner(n):
    nchunks = n // 16
    mesh = plsc.VectorSubcoreMesh(core_axis_name="c", subcore_axis_name="s")

    @functools.partial(
        pl.kernel, mesh=mesh,
        out_type=[jax.ShapeDtypeStruct((NT, 16, n), jnp.float32),
                  jax.ShapeDtypeStruct((_NW, 16), jnp.int32)],
        scratch_types=[pltpu.VMEM((16, n), jnp.float32),
                       pltpu.VMEM((4, n), jnp.int32),
                       pltpu.VMEM((1, n), jnp.int32),
                       pltpu.VMEM((16, n), jnp.float32),
                       pltpu.VMEM((16,), jnp.int32)],
        compiler_params=pltpu.CompilerParams(needs_layout_passes=False),
    )
    def binner(tablet_hbm, bounds_hbm, order_hbm, gath_out, cnt_out,
               table_v, bounds_v, order_v, rows_v, cnt_v):
        wid = lax.axis_index("s") * _NC + lax.axis_index("c")
        pltpu.sync_copy(tablet_hbm, table_v)
        pltpu.sync_copy(bounds_hbm, bounds_v)
        pltpu.sync_copy(order_hbm, order_v)
        iota = lax.broadcasted_iota(jnp.int32, (16,), 0)
        lane15 = jnp.full((16,), 15, jnp.int32)
        row_ids = [jnp.full((16,), r, jnp.int32) for r in range(11)]
        ones16 = jnp.ones((16,), jnp.float32)
        counts_vec = jnp.zeros((16,), jnp.int32)
        for k in range(_TPW):
            t = wid * _TPW + k
            ty = t // TW
            tx = t % TW
            base = jnp.zeros((16,), jnp.int32)
            for j in range(nchunks):
                ordc = order_v[0, pl.ds(j * 16, 16)]
                txm = plsc.load_gather(bounds_v, [row_ids[0], ordc])
                txM = plsc.load_gather(bounds_v, [row_ids[1], ordc])
                tym = plsc.load_gather(bounds_v, [row_ids[2], ordc])
                tyM = plsc.load_gather(bounds_v, [row_ids[3], ordc])
                m = (txm <= tx) & (tx <= txM) & (tym <= ty) & (ty <= tyM)
                cum = plsc.cumsum(jnp.where(m, 1, 0))
                pos = base + cum - 1
                for p in range(10):
                    vals = plsc.load_gather(table_v, [row_ids[p], ordc])
                    plsc.store_scatter(rows_v, [row_ids[p], pos], vals, mask=m)
                plsc.store_scatter(rows_v, [row_ids[10], pos], ones16, mask=m)
                base = base + lax.gather(
                    cum, lane15[:, None],
                    lax.GatherDimensionNumbers(offset_dims=(),
                                               collapsed_slice_dims=(0,),
                                               start_index_map=(0,)),
                    (1,), mode=lax.GatherScatterMode.PROMISE_IN_BOUNDS)
            pltpu.sync_copy(rows_v, gath_out.at[t])
            counts_vec = jnp.where(iota == k, base, counts_vec)
        cnt_v[...] = counts_vec
        pltpu.sync_copy(cnt_v, cnt_out.at[wid])

    return binner


def _raster_kernel(cnt_ref, p_ref, o_ref):
    t = pl.program_id(0)
    tyf = (t // TW).astype(jnp.float32)
    txf = (t % TW).astype(jnp.float32)
    idx = lax.broadcasted_iota(jnp.int32, (P, 1), 0)
    pyf = tyf * TILE + (idx // TILE).astype(jnp.float32)
    pxf = txf * TILE + (idx % TILE).astype(jnp.float32)
    cnt = cnt_ref[t]
    nch = (cnt + _G - 1) // _G
    lane = lax.broadcasted_iota(jnp.int32, (1, _G), 1)

    def body(c, carry):
        acc, tcar = carry
        sl = pl.ds(c * _G, _G)
        inb = (c * _G + lane) < cnt
        mx = p_ref[0, 0:1, sl]
        my = p_ref[0, 1:2, sl]
        ci00 = p_ref[0, 2:3, sl]
        cs = p_ref[0, 3:4, sl]
        ci11 = p_ref[0, 4:5, sl]
        op = p_ref[0, 5:6, sl]
        dx = pxf - mx
        dy = pyf - my
        mahal = dx * dx * ci00 + dx * dy * cs + dy * dy * ci11
        w = jnp.exp(-0.5 * mahal) * op
        alpha = jnp.where((w > 0.01) & inb, w, 0.0)
        cp = 1.0 - alpha
        k = 1
        while k < _G:
            o = jnp.ones((P, k), jnp.float32)
            cp = cp * jnp.concatenate([o, cp[:, :-k]], axis=1)
            k *= 2
        o1 = jnp.ones((P, 1), jnp.float32)
        texcl = jnp.concatenate([o1, cp[:, :-1]], axis=1) * tcar
        contrib = texcl * alpha
        p8 = jnp.where(inb, p_ref[0, 6:14, sl], 0.0)
        acc = acc + lax.dot_general(p8, contrib,
                                    (((1,), (1,)), ((), ())),
                                    preferred_element_type=jnp.float32)
        tcar = tcar * cp[:, -1:]
        return acc, tcar

    acc, _ = lax.fori_loop(
        0, nch, body,
        (jnp.zeros((8, P), jnp.float32), jnp.ones((P, 1), jnp.float32)))
    o_ref[0] = acc


def kernel(positions, scales, rotations, colors, opacities, K, background):
    n = positions.shape[0]
    inp = jnp.concatenate([positions.T, scales.T, rotations.T,
                           colors.T, opacities.T], axis=0).astype(jnp.float32)
    kvec = jnp.stack([K[0, 0], K[1, 1], K[0, 2], K[1, 2]]).astype(jnp.float32)
    table, bounds, order = pl.pallas_call(
        _prep_kernel,
        in_specs=[pl.BlockSpec(memory_space=pltpu.VMEM),
                  pl.BlockSpec(memory_space=pltpu.SMEM)],
        out_specs=[pl.BlockSpec(memory_space=pltpu.VMEM),
                   pl.BlockSpec(memory_space=pltpu.VMEM),
                   pl.BlockSpec(memory_space=pltpu.VMEM)],
        out_shape=[jax.ShapeDtypeStruct((16, n), jnp.float32),
                   jax.ShapeDtypeStruct((4, n), jnp.int32),
                   jax.ShapeDtypeStruct((1, n), jnp.int32)],
    )(inp, kvec)
    gathered, cnt32 = _make_sc_binner(n)(table, bounds, order)
    counts = cnt32[:, :_TPW].reshape(-1)
    grid_spec = pltpu.PrefetchScalarGridSpec(
        num_scalar_prefetch=1,
        grid=(NT,),
        in_specs=[pl.BlockSpec((1, 16, n), lambda t, s: (t, 0, 0))],
        out_specs=pl.BlockSpec((1, 8, P), lambda t, s: (t, 0, 0)),
    )
    out = pl.pallas_call(
        _raster_kernel,
        grid_spec=grid_spec,
        out_shape=jax.ShapeDtypeStruct((NT, 8, P), jnp.float32),
    )(counts, gathered)
    r = out.reshape(TH, TW, 8, TILE, TILE).transpose(2, 0, 3, 1, 4).reshape(8, H, W)
    img = jnp.clip(background[None, None, :] + r[0:3].transpose(1, 2, 0), 0.0, 1.0)
    return img, r[3], r[4]


# speedup vs baseline: 69.5534x; 1.0115x over previous
import functools

import jax
import jax.numpy as jnp
from jax import lax
from jax.experimental import pallas as pl
from jax.experimental.pallas import tpu as pltpu
from jax.experimental.pallas import tpu_sc as plsc

H, W = 128, 128
TILE = 16
TH = H // TILE
TW = W // TILE
NT = TH * TW
P = TILE * TILE

_NC, _NS = 2, 16
_NW = _NC * _NS
_TPW = NT // _NW
_G = 128


def _rolll(x, j):
    return jnp.concatenate([x[:, j:], x[:, :j]], axis=1)


def _rollr(x, j):
    return jnp.concatenate([x[:, -j:], x[:, :-j]], axis=1)


def _prep_kernel(inp_ref, k_ref, table_ref, bounds_ref, order_ref):
    n = inp_ref.shape[1]
    fx = k_ref[0]
    fy = k_ref[1]
    cx = k_ref[2]
    cy = k_ref[3]
    x = inp_ref[0:1, :]
    y = inp_ref[1:2, :]
    z = jnp.maximum(inp_ref[2:3, :], 1e-3)
    s0 = inp_ref[3:4, :]
    s1 = inp_ref[4:5, :]
    s2 = inp_ref[5:6, :]
    qw = inp_ref[6:7, :]
    qx = inp_ref[7:8, :]
    qy = inp_ref[8:9, :]
    qz = inp_ref[9:10, :]
    qn = jnp.sqrt(qw * qw + qx * qx + qy * qy + qz * qz) + 1e-8
    qw = qw / qn; qx = qx / qn; qy = qy / qn; qz = qz / qn
    r00 = 1 - 2 * (qy * qy + qz * qz); r01 = 2 * (qx * qy - qw * qz); r02 = 2 * (qx * qz + qw * qy)
    r10 = 2 * (qx * qy + qw * qz); r11 = 1 - 2 * (qx * qx + qz * qz); r12 = 2 * (qy * qz - qw * qx)
    r20 = 2 * (qx * qz - qw * qy); r21 = 2 * (qy * qz + qw * qx); r22 = 1 - 2 * (qx * qx + qy * qy)
    t0 = s0 * s0; t1 = s1 * s1; t2 = s2 * s2
    s00 = r00 * r00 * t0 + r01 * r01 * t1 + r02 * r02 * t2
    s01 = r00 * r10 * t0 + r01 * r11 * t1 + r02 * r12 * t2
    s02 = r00 * r20 * t0 + r01 * r21 * t1 + r02 * r22 * t2
    s11 = r10 * r10 * t0 + r11 * r11 * t1 + r12 * r12 * t2
    s12 = r10 * r20 * t0 + r11 * r21 * t1 + r12 * r22 * t2
    s22 = r20 * r20 * t0 + r21 * r21 * t1 + r22 * r22 * t2
    mx = fx * x / z + cx
    my = fy * y / z + cy
    a1 = fx / z
    b2 = fy / z
    c1 = -fx * x / (z * z)
    c2 = -fy * y / (z * z)
    cov00 = a1 * a1 * s00 + 2 * a1 * c1 * s02 + c1 * c1 * s22 + 1e-4
    cov01 = a1 * b2 * s01 + a1 * c2 * s02 + c1 * b2 * s12 + c1 * c2 * s22
    cov11 = b2 * b2 * s11 + 2 * b2 * c2 * s12 + c2 * c2 * s22 + 1e-4
    trace = cov00 + cov11
    det = cov00 * cov11 - cov01 * cov01
    lam = (trace + jnp.sqrt(jnp.maximum(trace * trace - 4 * det, 0.0))) * 0.5
    radius = 3.0 * jnp.sqrt(lam + 1e-6)
    ftw = jnp.float32(TW - 1)
    fth = jnp.float32(TH - 1)
    txmin = jnp.clip(jnp.floor((mx - radius) * (1.0 / TILE)), 0.0, ftw).astype(jnp.int32)
    txmax = jnp.clip(jnp.ceil((mx + radius) * (1.0 / TILE)), 0.0, ftw).astype(jnp.int32)
    tymin = jnp.clip(jnp.floor((my - radius) * (1.0 / TILE)), 0.0, fth).astype(jnp.int32)
    tymax = jnp.clip(jnp.ceil((my + radius) * (1.0 / TILE)), 0.0, fth).astype(jnp.int32)
    inv_det = 1.0 / (cov00 * cov11 - cov01 * cov01 + 1e-12)
    ci00 = cov11 * inv_det
    cs = -(cov01 + cov01) * inv_det
    ci11 = cov00 * inv_det
    key = -z
    pay = lax.broadcasted_iota(jnp.int32, (1, n), 1)
    li = lax.broadcasted_iota(jnp.int32, (1, n), 1)
    kk = 2
    while kk <= n:
        jj = kk // 2
        while jj >= 1:
            high = (li & jj) != 0
            desc = (li & kk) != 0
            tk = jnp.where(high, _rollr(key, jj), _rolll(key, jj))
            tp = jnp.where(high, _rollr(pay, jj), _rolll(pay, jj))
            lexgt = (key > tk) | ((key == tk) & (pay > tp))
            take = lexgt ^ high ^ desc
            key = jnp.where(take, tk, key)
            pay = jnp.where(take, tp, pay)
            jj //= 2
        kk *= 2
    order_ref[...] = pay
    table_ref[0:1, :] = mx
    table_ref[1:2, :] = my
    table_ref[2:3, :] = ci00
    table_ref[3:4, :] = cs
    table_ref[4:5, :] = ci11
    table_ref[5:6, :] = inp_ref[13:14, :]
    table_ref[6:9, :] = inp_ref[10:13, :]
    table_ref[9:10, :] = z
    table_ref[10:11, :] = jnp.ones((1, n), jnp.float32)
    table_ref[11:16, :] = jnp.zeros((5, n), jnp.float32)
    bounds_ref[0:1, :] = txmin
    bounds_ref[1:2, :] = txmax
    bounds_ref[2:3, :] = tymin
    bounds_ref[3:4, :] = tymax


def _make_sc_binner(n):
    nchunks = n // 16
    mesh = plsc.VectorSubcoreMesh(core_axis_name="c", subcore_axis_name="s")

    @functools.partial(
        pl.kernel, mesh=mesh,
        out_type=[jax.ShapeDtypeStruct((NT, 16, n), jnp.float32),
                  jax.ShapeDtypeStruct((_NW, 16), jnp.int32)],
        scratch_types=[pltpu.VMEM((16, n), jnp.float32),
                       pltpu.VMEM((4, n), jnp.int32),
                       pltpu.VMEM((1, n), jnp.int32),
                       pltpu.VMEM((16, n), jnp.float32),
                       pltpu.VMEM((16,), jnp.int32)],
        compiler_params=pltpu.CompilerParams(needs_layout_passes=False),
    )
    def binner(tablet_hbm, bounds_hbm, order_hbm, gath_out, cnt_out,
               table_v, bounds_v, order_v, rows_v, cnt_v):
        wid = lax.axis_index("s") * _NC + lax.axis_index("c")
        pltpu.sync_copy(tablet_hbm, table_v)
        pltpu.sync_copy(bounds_hbm, bounds_v)
        pltpu.sync_copy(order_hbm, order_v)
        iota = lax.broadcasted_iota(jnp.int32, (16,), 0)
        lane15 = jnp.full((16,), 15, jnp.int32)
        row_ids = [jnp.full((16,), r, jnp.int32) for r in range(11)]
        ones16 = jnp.ones((16,), jnp.float32)
        counts_vec = jnp.zeros((16,), jnp.int32)
        for k in range(_TPW):
            t = wid * _TPW + k
            ty = t // TW
            tx = t % TW
            base = jnp.zeros((16,), jnp.int32)
            for j in range(nchunks):
                ordc = order_v[0, pl.ds(j * 16, 16)]
                txm = plsc.load_gather(bounds_v, [row_ids[0], ordc])
                txM = plsc.load_gather(bounds_v, [row_ids[1], ordc])
                tym = plsc.load_gather(bounds_v, [row_ids[2], ordc])
                tyM = plsc.load_gather(bounds_v, [row_ids[3], ordc])
                m = (txm <= tx) & (tx <= txM) & (tym <= ty) & (ty <= tyM)
                cum = plsc.cumsum(jnp.where(m, 1, 0))
                pos = base + cum - 1
                for p in range(10):
                    vals = plsc.load_gather(table_v, [row_ids[p], ordc])
                    plsc.store_scatter(rows_v, [row_ids[p], pos], vals, mask=m)
                plsc.store_scatter(rows_v, [row_ids[10], pos], ones16, mask=m)
                base = base + lax.gather(
                    cum, lane15[:, None],
                    lax.GatherDimensionNumbers(offset_dims=(),
                                               collapsed_slice_dims=(0,),
                                               start_index_map=(0,)),
                    (1,), mode=lax.GatherScatterMode.PROMISE_IN_BOUNDS)
            pltpu.sync_copy(rows_v, gath_out.at[t])
            counts_vec = jnp.where(iota == k, base, counts_vec)
        cnt_v[...] = counts_vec
        pltpu.sync_copy(cnt_v, cnt_out.at[wid])

    return binner


_TB = 8


def _raster_kernel(cnt_ref, p_ref, o_ref):
    pid = pl.program_id(0)
    idx = lax.broadcasted_iota(jnp.int32, (P, 1), 0)
    iyf = (idx // TILE).astype(jnp.float32)
    ixf = (idx % TILE).astype(jnp.float32)
    lane = lax.broadcasted_iota(jnp.int32, (1, _G), 1)

    for tt in range(_TB):
        t = pid * _TB + tt
        tyf = (t // TW).astype(jnp.float32)
        txf = (t % TW).astype(jnp.float32)
        pyf = tyf * TILE + iyf
        pxf = txf * TILE + ixf
        cnt = cnt_ref[t]
        nch = (cnt + _G - 1) // _G

        def body(c, carry, tt=tt, cnt=cnt, pxf=pxf, pyf=pyf):
            acc, tcar = carry
            sl = pl.ds(c * _G, _G)
            inb = (c * _G + lane) < cnt
            mx = p_ref[tt, 0:1, sl]
            my = p_ref[tt, 1:2, sl]
            ci00 = p_ref[tt, 2:3, sl]
            cs = p_ref[tt, 3:4, sl]
            ci11 = p_ref[tt, 4:5, sl]
            op = p_ref[tt, 5:6, sl]
            dx = pxf - mx
            dy = pyf - my
            mahal = dx * dx * ci00 + dx * dy * cs + dy * dy * ci11
            w = jnp.exp(-0.5 * mahal) * op
            alpha = jnp.where((w > 0.01) & inb, w, 0.0)
            cp = 1.0 - alpha
            k = 1
            while k < _G:
                o = jnp.ones((P, k), jnp.float32)
                cp = cp * jnp.concatenate([o, cp[:, :-k]], axis=1)
                k *= 2
            o1 = jnp.ones((P, 1), jnp.float32)
            texcl = jnp.concatenate([o1, cp[:, :-1]], axis=1) * tcar
            contrib = texcl * alpha
            p8 = jnp.where(inb, p_ref[tt, 6:14, sl], 0.0)
            acc = acc + lax.dot_general(p8, contrib,
                                        (((1,), (1,)), ((), ())),
                                        preferred_element_type=jnp.float32)
            tcar = tcar * cp[:, -1:]
            return acc, tcar

        acc, _ = lax.fori_loop(
            0, nch, body,
            (jnp.zeros((8, P), jnp.float32), jnp.ones((P, 1), jnp.float32)))
        o_ref[tt] = acc


def kernel(positions, scales, rotations, colors, opacities, K, background):
    n = positions.shape[0]
    inp = jnp.concatenate([positions.T, scales.T, rotations.T,
                           colors.T, opacities.T], axis=0).astype(jnp.float32)
    kvec = jnp.stack([K[0, 0], K[1, 1], K[0, 2], K[1, 2]]).astype(jnp.float32)
    table, bounds, order = pl.pallas_call(
        _prep_kernel,
        in_specs=[pl.BlockSpec(memory_space=pltpu.VMEM),
                  pl.BlockSpec(memory_space=pltpu.SMEM)],
        out_specs=[pl.BlockSpec(memory_space=pltpu.VMEM),
                   pl.BlockSpec(memory_space=pltpu.VMEM),
                   pl.BlockSpec(memory_space=pltpu.VMEM)],
        out_shape=[jax.ShapeDtypeStruct((16, n), jnp.float32),
                   jax.ShapeDtypeStruct((4, n), jnp.int32),
                   jax.ShapeDtypeStruct((1, n), jnp.int32)],
    )(inp, kvec)
    gathered, cnt32 = _make_sc_binner(n)(table, bounds, order)
    counts = cnt32[:, :_TPW].reshape(-1)
    grid_spec = pltpu.PrefetchScalarGridSpec(
        num_scalar_prefetch=1,
        grid=(NT // _TB,),
        in_specs=[pl.BlockSpec((_TB, 16, n), lambda t, s: (t, 0, 0))],
        out_specs=pl.BlockSpec((_TB, 8, P), lambda t, s: (t, 0, 0)),
    )
    out = pl.pallas_call(
        _raster_kernel,
        grid_spec=grid_spec,
        out_shape=jax.ShapeDtypeStruct((NT, 8, P), jnp.float32),
    )(counts, gathered)
    r = out.reshape(TH, TW, 8, TILE, TILE).transpose(2, 0, 3, 1, 4).reshape(8, H, W)
    img = jnp.clip(background[None, None, :] + r[0:3].transpose(1, 2, 0), 0.0, 1.0)
    return img, r[3], r[4]


# speedup vs baseline: 90.2335x; 1.2973x over previous
import functools

import jax
import jax.numpy as jnp
from jax import lax
from jax.experimental import pallas as pl
from jax.experimental.pallas import tpu as pltpu
from jax.experimental.pallas import tpu_sc as plsc

H, W = 128, 128
TILE = 16
TH = H // TILE
TW = W // TILE
NT = TH * TW
P = TILE * TILE

_NC, _NS = 2, 16
_NW = _NC * _NS
_TPW = NT // _NW
_G = 128


def _rolll(x, j):
    return jnp.concatenate([x[:, j:], x[:, :j]], axis=1)


def _rollr(x, j):
    return jnp.concatenate([x[:, -j:], x[:, :-j]], axis=1)


def _prep_kernel(inp_ref, k_ref, table_ref, bounds_ref, order_ref):
    n = inp_ref.shape[1]
    fx = k_ref[0]
    fy = k_ref[1]
    cx = k_ref[2]
    cy = k_ref[3]
    x = inp_ref[0:1, :]
    y = inp_ref[1:2, :]
    z = jnp.maximum(inp_ref[2:3, :], 1e-3)
    s0 = inp_ref[3:4, :]
    s1 = inp_ref[4:5, :]
    s2 = inp_ref[5:6, :]
    qw = inp_ref[6:7, :]
    qx = inp_ref[7:8, :]
    qy = inp_ref[8:9, :]
    qz = inp_ref[9:10, :]
    qn = jnp.sqrt(qw * qw + qx * qx + qy * qy + qz * qz) + 1e-8
    qw = qw / qn; qx = qx / qn; qy = qy / qn; qz = qz / qn
    r00 = 1 - 2 * (qy * qy + qz * qz); r01 = 2 * (qx * qy - qw * qz); r02 = 2 * (qx * qz + qw * qy)
    r10 = 2 * (qx * qy + qw * qz); r11 = 1 - 2 * (qx * qx + qz * qz); r12 = 2 * (qy * qz - qw * qx)
    r20 = 2 * (qx * qz - qw * qy); r21 = 2 * (qy * qz + qw * qx); r22 = 1 - 2 * (qx * qx + qy * qy)
    t0 = s0 * s0; t1 = s1 * s1; t2 = s2 * s2
    s00 = r00 * r00 * t0 + r01 * r01 * t1 + r02 * r02 * t2
    s01 = r00 * r10 * t0 + r01 * r11 * t1 + r02 * r12 * t2
    s02 = r00 * r20 * t0 + r01 * r21 * t1 + r02 * r22 * t2
    s11 = r10 * r10 * t0 + r11 * r11 * t1 + r12 * r12 * t2
    s12 = r10 * r20 * t0 + r11 * r21 * t1 + r12 * r22 * t2
    s22 = r20 * r20 * t0 + r21 * r21 * t1 + r22 * r22 * t2
    mx = fx * x / z + cx
    my = fy * y / z + cy
    a1 = fx / z
    b2 = fy / z
    c1 = -fx * x / (z * z)
    c2 = -fy * y / (z * z)
    cov00 = a1 * a1 * s00 + 2 * a1 * c1 * s02 + c1 * c1 * s22 + 1e-4
    cov01 = a1 * b2 * s01 + a1 * c2 * s02 + c1 * b2 * s12 + c1 * c2 * s22
    cov11 = b2 * b2 * s11 + 2 * b2 * c2 * s12 + c2 * c2 * s22 + 1e-4
    trace = cov00 + cov11
    det = cov00 * cov11 - cov01 * cov01
    lam = (trace + jnp.sqrt(jnp.maximum(trace * trace - 4 * det, 0.0))) * 0.5
    radius = 3.0 * jnp.sqrt(lam + 1e-6)
    ftw = jnp.float32(TW - 1)
    fth = jnp.float32(TH - 1)
    txmin = jnp.clip(jnp.floor((mx - radius) * (1.0 / TILE)), 0.0, ftw).astype(jnp.int32)
    txmax = jnp.clip(jnp.ceil((mx + radius) * (1.0 / TILE)), 0.0, ftw).astype(jnp.int32)
    tymin = jnp.clip(jnp.floor((my - radius) * (1.0 / TILE)), 0.0, fth).astype(jnp.int32)
    tymax = jnp.clip(jnp.ceil((my + radius) * (1.0 / TILE)), 0.0, fth).astype(jnp.int32)
    one32 = jnp.int32(1)
    zero_row = jnp.zeros((1, n), jnp.int32)
    lo = zero_row
    hi = zero_row
    for t in range(NT):
        ty_t = t // TW
        tx_t = t % TW
        cov = ((txmin <= tx_t) & (tx_t <= txmax)
               & (tymin <= ty_t) & (ty_t <= tymax))
        bit = jnp.left_shift(one32, t % 32)
        if t < 32:
            lo = lo | jnp.where(cov, bit, 0)
        else:
            hi = hi | jnp.where(cov, bit, 0)
    inv_det = 1.0 / (cov00 * cov11 - cov01 * cov01 + 1e-12)
    ci00 = cov11 * inv_det
    cs = -(cov01 + cov01) * inv_det
    ci11 = cov00 * inv_det
    key = -z
    pay = lax.broadcasted_iota(jnp.int32, (1, n), 1)
    li = lax.broadcasted_iota(jnp.int32, (1, n), 1)
    kk = 2
    while kk <= n:
        jj = kk // 2
        while jj >= 1:
            high = (li & jj) != 0
            desc = (li & kk) != 0
            tk = jnp.where(high, _rollr(key, jj), _rolll(key, jj))
            tp = jnp.where(high, _rollr(pay, jj), _rolll(pay, jj))
            lexgt = (key > tk) | ((key == tk) & (pay > tp))
            take = lexgt ^ high ^ desc
            key = jnp.where(take, tk, key)
            pay = jnp.where(take, tp, pay)
            jj //= 2
        kk *= 2
    order_ref[...] = pay
    table_ref[0:1, :] = mx
    table_ref[1:2, :] = my
    table_ref[2:3, :] = ci00
    table_ref[3:4, :] = cs
    table_ref[4:5, :] = ci11
    table_ref[5:6, :] = inp_ref[13:14, :]
    table_ref[6:9, :] = inp_ref[10:13, :]
    table_ref[9:10, :] = z
    table_ref[10:11, :] = jnp.ones((1, n), jnp.float32)
    table_ref[11:16, :] = jnp.zeros((5, n), jnp.float32)
    bounds_ref[0:1, :] = lo
    bounds_ref[1:2, :] = hi


def _make_sc_binner(n):
    nchunks = n // 16
    mesh = plsc.VectorSubcoreMesh(core_axis_name="c", subcore_axis_name="s")

    @functools.partial(
        pl.kernel, mesh=mesh,
        out_type=[jax.ShapeDtypeStruct((NT, 16, n), jnp.float32),
                  jax.ShapeDtypeStruct((_NW, 16), jnp.int32)],
        scratch_types=[pltpu.VMEM((16, n), jnp.float32),
                       pltpu.VMEM((2, n), jnp.int32),
                       pltpu.VMEM((1, n), jnp.int32),
                       pltpu.VMEM((16, n), jnp.float32),
                       pltpu.VMEM((16,), jnp.int32)],
        compiler_params=pltpu.CompilerParams(needs_layout_passes=False),
    )
    def binner(tablet_hbm, bounds_hbm, order_hbm, gath_out, cnt_out,
               table_v, bounds_v, order_v, rows_v, cnt_v):
        wid = lax.axis_index("s") * _NC + lax.axis_index("c")
        pltpu.sync_copy(tablet_hbm, table_v)
        pltpu.sync_copy(bounds_hbm, bounds_v)
        pltpu.sync_copy(order_hbm, order_v)
        iota = lax.broadcasted_iota(jnp.int32, (16,), 0)
        lane15 = jnp.full((16,), 15, jnp.int32)
        row_ids = [jnp.full((16,), r, jnp.int32) for r in range(11)]
        ones16 = jnp.ones((16,), jnp.float32)
        counts_vec = jnp.zeros((16,), jnp.int32)
        for k in range(_TPW):
            t = wid * _TPW + k
            base = jnp.zeros((16,), jnp.int32)
            wrow = jnp.zeros((16,), jnp.int32) + (t // 32)
            bit = t % 32
            for j in range(nchunks):
                ordc = order_v[0, pl.ds(j * 16, 16)]
                wc = plsc.load_gather(bounds_v, [wrow, ordc])
                m = (lax.shift_right_logical(wc, bit) & 1) != 0
                cum = plsc.cumsum(jnp.where(m, 1, 0))
                pos = base + cum - 1
                for p in range(10):
                    vals = plsc.load_gather(table_v, [row_ids[p], ordc])
                    plsc.store_scatter(rows_v, [row_ids[p], pos], vals, mask=m)
                plsc.store_scatter(rows_v, [row_ids[10], pos], ones16, mask=m)
                base = base + lax.gather(
                    cum, lane15[:, None],
                    lax.GatherDimensionNumbers(offset_dims=(),
                                               collapsed_slice_dims=(0,),
                                               start_index_map=(0,)),
                    (1,), mode=lax.GatherScatterMode.PROMISE_IN_BOUNDS)
            pltpu.sync_copy(rows_v, gath_out.at[t])
            counts_vec = jnp.where(iota == k, base, counts_vec)
        cnt_v[...] = counts_vec
        pltpu.sync_copy(cnt_v, cnt_out.at[wid])

    return binner


_TB = 8


def _raster_kernel(cnt_ref, p_ref, o_ref):
    pid = pl.program_id(0)
    idx = lax.broadcasted_iota(jnp.int32, (P, 1), 0)
    iyf = (idx // TILE).astype(jnp.float32)
    ixf = (idx % TILE).astype(jnp.float32)
    lane = lax.broadcasted_iota(jnp.int32, (1, _G), 1)
    ur = lax.broadcasted_iota(jnp.int32, (_G, 1), 0)
    uc = lax.broadcasted_iota(jnp.int32, (1, _G), 1)
    ut = jnp.where(ur < uc, 1.0, 0.0)

    for tt in range(_TB):
        t = pid * _TB + tt
        tyf = (t // TW).astype(jnp.float32)
        txf = (t % TW).astype(jnp.float32)
        pyf = tyf * TILE + iyf
        pxf = txf * TILE + ixf
        cnt = cnt_ref[t]
        nch = (cnt + _G - 1) // _G

        def body(c, carry, tt=tt, cnt=cnt, pxf=pxf, pyf=pyf):
            acc, tcar = carry
            sl = pl.ds(c * _G, _G)
            inb = (c * _G + lane) < cnt
            mx = p_ref[tt, 0:1, sl]
            my = p_ref[tt, 1:2, sl]
            ci00 = p_ref[tt, 2:3, sl]
            cs = p_ref[tt, 3:4, sl]
            ci11 = p_ref[tt, 4:5, sl]
            op = p_ref[tt, 5:6, sl]
            dx = pxf - mx
            dy = pyf - my
            mahal = dx * dx * ci00 + dx * dy * cs + dy * dy * ci11
            w = jnp.exp(-0.5 * mahal) * op
            alpha = jnp.where((w > 0.01) & inb, w, 0.0)
            lg = jnp.log(1.0 - alpha)
            cume = lax.dot_general(lg, ut, (((1,), (0,)), ((), ())),
                                   preferred_element_type=jnp.float32)
            texcl = jnp.exp(cume) * tcar
            contrib = texcl * alpha
            p8 = jnp.where(inb, p_ref[tt, 6:14, sl], 0.0)
            acc = acc + lax.dot_general(p8, contrib,
                                        (((1,), (1,)), ((), ())),
                                        preferred_element_type=jnp.float32)
            tcar = tcar * jnp.exp(jnp.sum(lg, axis=1, keepdims=True))
            return acc, tcar

        acc, _ = lax.fori_loop(
            0, nch, body,
            (jnp.zeros((8, P), jnp.float32), jnp.ones((P, 1), jnp.float32)))
        o_ref[tt] = acc


def kernel(positions, scales, rotations, colors, opacities, K, background):
    n = positions.shape[0]
    inp = jnp.concatenate([positions.T, scales.T, rotations.T,
                           colors.T, opacities.T], axis=0).astype(jnp.float32)
    kvec = jnp.stack([K[0, 0], K[1, 1], K[0, 2], K[1, 2]]).astype(jnp.float32)
    table, bounds, order = pl.pallas_call(
        _prep_kernel,
        in_specs=[pl.BlockSpec(memory_space=pltpu.VMEM),
                  pl.BlockSpec(memory_space=pltpu.SMEM)],
        out_specs=[pl.BlockSpec(memory_space=pltpu.VMEM),
                   pl.BlockSpec(memory_space=pltpu.VMEM),
                   pl.BlockSpec(memory_space=pltpu.VMEM)],
        out_shape=[jax.ShapeDtypeStruct((16, n), jnp.float32),
                   jax.ShapeDtypeStruct((2, n), jnp.int32),
                   jax.ShapeDtypeStruct((1, n), jnp.int32)],
    )(inp, kvec)
    gathered, cnt32 = _make_sc_binner(n)(table, bounds, order)
    counts = cnt32[:, :_TPW].reshape(-1)
    grid_spec = pltpu.PrefetchScalarGridSpec(
        num_scalar_prefetch=1,
        grid=(NT // _TB,),
        in_specs=[pl.BlockSpec((_TB, 16, n), lambda t, s: (t, 0, 0))],
        out_specs=pl.BlockSpec((_TB, 8, P), lambda t, s: (t, 0, 0)),
    )
    out = pl.pallas_call(
        _raster_kernel,
        grid_spec=grid_spec,
        out_shape=jax.ShapeDtypeStruct((NT, 8, P), jnp.float32),
    )(counts, gathered)
    r = out.reshape(TH, TW, 8, TILE, TILE).transpose(2, 0, 3, 1, 4).reshape(8, H, W)
    img = jnp.clip(background[None, None, :] + r[0:3].transpose(1, 2, 0), 0.0, 1.0)
    return img, r[3], r[4]


# speedup vs baseline: 94.1869x; 1.0438x over previous
import functools

import jax
import jax.numpy as jnp
from jax import lax
from jax.experimental import pallas as pl
from jax.experimental.pallas import tpu as pltpu
from jax.experimental.pallas import tpu_sc as plsc

H, W = 128, 128
TILE = 16
TH = H // TILE
TW = W // TILE
NT = TH * TW
P = TILE * TILE

_NC, _NS = 2, 16
_NW = _NC * _NS
_TPW = NT // _NW
_G = 128


def _rolll(x, j):
    return jnp.concatenate([x[:, j:], x[:, :j]], axis=1)


def _rollr(x, j):
    return jnp.concatenate([x[:, -j:], x[:, :-j]], axis=1)


def _prep_kernel(inp_ref, k_ref, table_ref, bounds_ref, order_ref):
    n = inp_ref.shape[1]
    fx = k_ref[0]
    fy = k_ref[1]
    cx = k_ref[2]
    cy = k_ref[3]
    x = inp_ref[0:1, :]
    y = inp_ref[1:2, :]
    z = jnp.maximum(inp_ref[2:3, :], 1e-3)
    s0 = inp_ref[3:4, :]
    s1 = inp_ref[4:5, :]
    s2 = inp_ref[5:6, :]
    qw = inp_ref[6:7, :]
    qx = inp_ref[7:8, :]
    qy = inp_ref[8:9, :]
    qz = inp_ref[9:10, :]
    qn = jnp.sqrt(qw * qw + qx * qx + qy * qy + qz * qz) + 1e-8
    qw = qw / qn; qx = qx / qn; qy = qy / qn; qz = qz / qn
    r00 = 1 - 2 * (qy * qy + qz * qz); r01 = 2 * (qx * qy - qw * qz); r02 = 2 * (qx * qz + qw * qy)
    r10 = 2 * (qx * qy + qw * qz); r11 = 1 - 2 * (qx * qx + qz * qz); r12 = 2 * (qy * qz - qw * qx)
    r20 = 2 * (qx * qz - qw * qy); r21 = 2 * (qy * qz + qw * qx); r22 = 1 - 2 * (qx * qx + qy * qy)
    t0 = s0 * s0; t1 = s1 * s1; t2 = s2 * s2
    s00 = r00 * r00 * t0 + r01 * r01 * t1 + r02 * r02 * t2
    s01 = r00 * r10 * t0 + r01 * r11 * t1 + r02 * r12 * t2
    s02 = r00 * r20 * t0 + r01 * r21 * t1 + r02 * r22 * t2
    s11 = r10 * r10 * t0 + r11 * r11 * t1 + r12 * r12 * t2
    s12 = r10 * r20 * t0 + r11 * r21 * t1 + r12 * r22 * t2
    s22 = r20 * r20 * t0 + r21 * r21 * t1 + r22 * r22 * t2
    mx = fx * x / z + cx
    my = fy * y / z + cy
    a1 = fx / z
    b2 = fy / z
    c1 = -fx * x / (z * z)
    c2 = -fy * y / (z * z)
    cov00 = a1 * a1 * s00 + 2 * a1 * c1 * s02 + c1 * c1 * s22 + 1e-4
    cov01 = a1 * b2 * s01 + a1 * c2 * s02 + c1 * b2 * s12 + c1 * c2 * s22
    cov11 = b2 * b2 * s11 + 2 * b2 * c2 * s12 + c2 * c2 * s22 + 1e-4
    trace = cov00 + cov11
    det = cov00 * cov11 - cov01 * cov01
    lam = (trace + jnp.sqrt(jnp.maximum(trace * trace - 4 * det, 0.0))) * 0.5
    radius = 3.0 * jnp.sqrt(lam + 1e-6)
    ftw = jnp.float32(TW - 1)
    fth = jnp.float32(TH - 1)
    txmin = jnp.clip(jnp.floor((mx - radius) * (1.0 / TILE)), 0.0, ftw).astype(jnp.int32)
    txmax = jnp.clip(jnp.ceil((mx + radius) * (1.0 / TILE)), 0.0, ftw).astype(jnp.int32)
    tymin = jnp.clip(jnp.floor((my - radius) * (1.0 / TILE)), 0.0, fth).astype(jnp.int32)
    tymax = jnp.clip(jnp.ceil((my + radius) * (1.0 / TILE)), 0.0, fth).astype(jnp.int32)
    one32 = jnp.int32(1)
    zero_row = jnp.zeros((1, n), jnp.int32)
    lo = zero_row
    hi = zero_row
    for t in range(NT):
        ty_t = t // TW
        tx_t = t % TW
        cov = ((txmin <= tx_t) & (tx_t <= txmax)
               & (tymin <= ty_t) & (ty_t <= tymax))
        bit = jnp.left_shift(one32, t % 32)
        if t < 32:
            lo = lo | jnp.where(cov, bit, 0)
        else:
            hi = hi | jnp.where(cov, bit, 0)
    inv_det = 1.0 / (cov00 * cov11 - cov01 * cov01 + 1e-12)
    ci00 = cov11 * inv_det
    cs = -(cov01 + cov01) * inv_det
    ci11 = cov00 * inv_det
    key = -z
    pay = lax.broadcasted_iota(jnp.int32, (1, n), 1)
    li = lax.broadcasted_iota(jnp.int32, (1, n), 1)
    kk = 2
    while kk <= n:
        jj = kk // 2
        while jj >= 1:
            high = (li & jj) != 0
            desc = (li & kk) != 0
            tk = jnp.where(high, _rollr(key, jj), _rolll(key, jj))
            tp = jnp.where(high, _rollr(pay, jj), _rolll(pay, jj))
            lexgt = (key > tk) | ((key == tk) & (pay > tp))
            take = lexgt ^ high ^ desc
            key = jnp.where(take, tk, key)
            pay = jnp.where(take, tp, pay)
            jj //= 2
        kk *= 2
    order_ref[...] = pay
    table_ref[0:1, :] = mx
    table_ref[1:2, :] = my
    table_ref[2:3, :] = ci00
    table_ref[3:4, :] = cs
    table_ref[4:5, :] = ci11
    table_ref[5:6, :] = inp_ref[13:14, :]
    table_ref[6:9, :] = inp_ref[10:13, :]
    table_ref[9:10, :] = z
    table_ref[10:11, :] = jnp.ones((1, n), jnp.float32)
    table_ref[11:16, :] = jnp.zeros((5, n), jnp.float32)
    bounds_ref[0:1, :] = lo
    bounds_ref[1:2, :] = hi


def _make_sc_binner(n):
    nchunks = n // 16
    mesh = plsc.VectorSubcoreMesh(core_axis_name="c", subcore_axis_name="s")

    @functools.partial(
        pl.kernel, mesh=mesh,
        out_type=[jax.ShapeDtypeStruct((NT, 16, n), jnp.float32),
                  jax.ShapeDtypeStruct((_NW, 16), jnp.int32)],
        scratch_types=[pltpu.VMEM((16, n), jnp.float32),
                       pltpu.VMEM((2, n), jnp.int32),
                       pltpu.VMEM((1, n), jnp.int32),
                       pltpu.VMEM((16, n), jnp.float32),
                       pltpu.VMEM((16, n), jnp.float32),
                       pltpu.VMEM((16,), jnp.int32)],
        compiler_params=pltpu.CompilerParams(needs_layout_passes=False),
    )
    def binner(tablet_hbm, bounds_hbm, order_hbm, gath_out, cnt_out,
               table_v, bounds_v, order_v, rows_v, rows2_v, cnt_v):
        wid = lax.axis_index("s") * _NC + lax.axis_index("c")
        pltpu.sync_copy(tablet_hbm, table_v)
        pltpu.sync_copy(bounds_hbm, bounds_v)
        pltpu.sync_copy(order_hbm, order_v)
        iota = lax.broadcasted_iota(jnp.int32, (16,), 0)
        lane15 = jnp.full((16,), 15, jnp.int32)
        row_ids = [jnp.full((16,), r, jnp.int32) for r in range(11)]
        ones16 = jnp.ones((16,), jnp.float32)
        gdn = lax.GatherDimensionNumbers(offset_dims=(),
                                         collapsed_slice_dims=(0,),
                                         start_index_map=(0,))
        t0 = wid * _TPW
        wrow = jnp.zeros((16,), jnp.int32) + (t0 // 32)
        bit0 = t0 % 32
        bit1 = (t0 + 1) % 32
        base0 = jnp.zeros((16,), jnp.int32)
        base1 = jnp.zeros((16,), jnp.int32)
        for j in range(nchunks):
            ordc = order_v[0, pl.ds(j * 16, 16)]
            wc = plsc.load_gather(bounds_v, [wrow, ordc])
            m0 = (lax.shift_right_logical(wc, bit0) & 1) != 0
            m1 = (lax.shift_right_logical(wc, bit1) & 1) != 0
            cum0 = plsc.cumsum(jnp.where(m0, 1, 0))
            cum1 = plsc.cumsum(jnp.where(m1, 1, 0))
            pos0 = base0 + cum0 - 1
            pos1 = base1 + cum1 - 1
            for p in range(10):
                vals = plsc.load_gather(table_v, [row_ids[p], ordc])
                plsc.store_scatter(rows_v, [row_ids[p], pos0], vals, mask=m0)
                plsc.store_scatter(rows2_v, [row_ids[p], pos1], vals, mask=m1)
            plsc.store_scatter(rows_v, [row_ids[10], pos0], ones16, mask=m0)
            plsc.store_scatter(rows2_v, [row_ids[10], pos1], ones16, mask=m1)
            base0 = base0 + lax.gather(cum0, lane15[:, None], gdn, (1,),
                                       mode=lax.GatherScatterMode.PROMISE_IN_BOUNDS)
            base1 = base1 + lax.gather(cum1, lane15[:, None], gdn, (1,),
                                       mode=lax.GatherScatterMode.PROMISE_IN_BOUNDS)
        pltpu.sync_copy(rows_v, gath_out.at[t0])
        pltpu.sync_copy(rows2_v, gath_out.at[t0 + 1])
        counts_vec = jnp.where(iota == 0, base0, jnp.zeros((16,), jnp.int32))
        counts_vec = jnp.where(iota == 1, base1, counts_vec)
        cnt_v[...] = counts_vec
        pltpu.sync_copy(cnt_v, cnt_out.at[wid])

    return binner


_TB = 8


def _raster_kernel(cnt_ref, p_ref, o_ref):
    pid = pl.program_id(0)
    idx = lax.broadcasted_iota(jnp.int32, (P, 1), 0)
    iyf = (idx // TILE).astype(jnp.float32)
    ixf = (idx % TILE).astype(jnp.float32)
    lane = lax.broadcasted_iota(jnp.int32, (1, _G), 1)
    ur = lax.broadcasted_iota(jnp.int32, (_G, 1), 0)
    uc = lax.broadcasted_iota(jnp.int32, (1, _G), 1)
    ut = jnp.where(ur < uc, 1.0, 0.0)

    for tt in range(_TB):
        t = pid * _TB + tt
        tyf = (t // TW).astype(jnp.float32)
        txf = (t % TW).astype(jnp.float32)
        pyf = tyf * TILE + iyf
        pxf = txf * TILE + ixf
        cnt = cnt_ref[t]
        nch = (cnt + _G - 1) // _G

        def body(c, carry, tt=tt, cnt=cnt, pxf=pxf, pyf=pyf):
            acc, tcar = carry
            sl = pl.ds(c * _G, _G)
            inb = (c * _G + lane) < cnt
            mx = p_ref[tt, 0:1, sl]
            my = p_ref[tt, 1:2, sl]
            ci00 = p_ref[tt, 2:3, sl]
            cs = p_ref[tt, 3:4, sl]
            ci11 = p_ref[tt, 4:5, sl]
            op = p_ref[tt, 5:6, sl]
            dx = pxf - mx
            dy = pyf - my
            mahal = dx * dx * ci00 + dx * dy * cs + dy * dy * ci11
            w = jnp.exp(-0.5 * mahal) * op
            alpha = jnp.where((w > 0.01) & inb, w, 0.0)
            lg = jnp.log(1.0 - alpha)
            cume = lax.dot_general(lg, ut, (((1,), (0,)), ((), ())),
                                   preferred_element_type=jnp.float32)
            texcl = jnp.exp(cume) * tcar
            contrib = texcl * alpha
            p8 = jnp.where(inb, p_ref[tt, 6:14, sl], 0.0)
            acc = acc + lax.dot_general(p8, contrib,
                                        (((1,), (1,)), ((), ())),
                                        preferred_element_type=jnp.float32)
            tcar = tcar * jnp.exp(jnp.sum(lg, axis=1, keepdims=True))
            return acc, tcar

        acc, _ = lax.fori_loop(
            0, nch, body,
            (jnp.zeros((8, P), jnp.float32), jnp.ones((P, 1), jnp.float32)))
        o_ref[tt] = acc


def kernel(positions, scales, rotations, colors, opacities, K, background):
    n = positions.shape[0]
    inp = jnp.concatenate([positions.T, scales.T, rotations.T,
                           colors.T, opacities.T], axis=0).astype(jnp.float32)
    kvec = jnp.stack([K[0, 0], K[1, 1], K[0, 2], K[1, 2]]).astype(jnp.float32)
    table, bounds, order = pl.pallas_call(
        _prep_kernel,
        in_specs=[pl.BlockSpec(memory_space=pltpu.VMEM),
                  pl.BlockSpec(memory_space=pltpu.SMEM)],
        out_specs=[pl.BlockSpec(memory_space=pltpu.VMEM),
                   pl.BlockSpec(memory_space=pltpu.VMEM),
                   pl.BlockSpec(memory_space=pltpu.VMEM)],
        out_shape=[jax.ShapeDtypeStruct((16, n), jnp.float32),
                   jax.ShapeDtypeStruct((2, n), jnp.int32),
                   jax.ShapeDtypeStruct((1, n), jnp.int32)],
    )(inp, kvec)
    gathered, cnt32 = _make_sc_binner(n)(table, bounds, order)
    counts = cnt32[:, :_TPW].reshape(-1)
    grid_spec = pltpu.PrefetchScalarGridSpec(
        num_scalar_prefetch=1,
        grid=(NT // _TB,),
        in_specs=[pl.BlockSpec((_TB, 16, n), lambda t, s: (t, 0, 0))],
        out_specs=pl.BlockSpec((_TB, 8, P), lambda t, s: (t, 0, 0)),
    )
    out = pl.pallas_call(
        _raster_kernel,
        grid_spec=grid_spec,
        out_shape=jax.ShapeDtypeStruct((NT, 8, P), jnp.float32),
    )(counts, gathered)
    r = out.reshape(TH, TW, 8, TILE, TILE).transpose(2, 0, 3, 1, 4).reshape(8, H, W)
    img = jnp.clip(background[None, None, :] + r[0:3].transpose(1, 2, 0), 0.0, 1.0)
    return img, r[3], r[4]


# speedup vs baseline: 96.8875x; 1.0287x over previous
import functools

import jax
import jax.numpy as jnp
from jax import lax
from jax.experimental import pallas as pl
from jax.experimental.pallas import tpu as pltpu
from jax.experimental.pallas import tpu_sc as plsc

H, W = 128, 128
TILE = 16
TH = H // TILE
TW = W // TILE
NT = TH * TW
P = TILE * TILE

_NC, _NS = 2, 16
_NW = _NC * _NS
_TPW = NT // _NW
_G = 128


def _rolll(x, j):
    return jnp.concatenate([x[:, j:], x[:, :j]], axis=1)


def _rollr(x, j):
    return jnp.concatenate([x[:, -j:], x[:, :-j]], axis=1)


def _rollls(x, j):
    return jnp.concatenate([x[j:, :], x[:j, :]], axis=0)


def _rollrs(x, j):
    return jnp.concatenate([x[-j:, :], x[:-j, :]], axis=0)


def _prep_kernel(inp_ref, k_ref, table_ref, bounds_ref, order_ref):
    n = inp_ref.shape[1]
    fx = k_ref[0]
    fy = k_ref[1]
    cx = k_ref[2]
    cy = k_ref[3]
    x = inp_ref[0:1, :]
    y = inp_ref[1:2, :]
    z = jnp.maximum(inp_ref[2:3, :], 1e-3)
    s0 = inp_ref[3:4, :]
    s1 = inp_ref[4:5, :]
    s2 = inp_ref[5:6, :]
    qw = inp_ref[6:7, :]
    qx = inp_ref[7:8, :]
    qy = inp_ref[8:9, :]
    qz = inp_ref[9:10, :]
    qn = jnp.sqrt(qw * qw + qx * qx + qy * qy + qz * qz) + 1e-8
    qw = qw / qn; qx = qx / qn; qy = qy / qn; qz = qz / qn
    r00 = 1 - 2 * (qy * qy + qz * qz); r01 = 2 * (qx * qy - qw * qz); r02 = 2 * (qx * qz + qw * qy)
    r10 = 2 * (qx * qy + qw * qz); r11 = 1 - 2 * (qx * qx + qz * qz); r12 = 2 * (qy * qz - qw * qx)
    r20 = 2 * (qx * qz - qw * qy); r21 = 2 * (qy * qz + qw * qx); r22 = 1 - 2 * (qx * qx + qy * qy)
    t0 = s0 * s0; t1 = s1 * s1; t2 = s2 * s2
    s00 = r00 * r00 * t0 + r01 * r01 * t1 + r02 * r02 * t2
    s01 = r00 * r10 * t0 + r01 * r11 * t1 + r02 * r12 * t2
    s02 = r00 * r20 * t0 + r01 * r21 * t1 + r02 * r22 * t2
    s11 = r10 * r10 * t0 + r11 * r11 * t1 + r12 * r12 * t2
    s12 = r10 * r20 * t0 + r11 * r21 * t1 + r12 * r22 * t2
    s22 = r20 * r20 * t0 + r21 * r21 * t1 + r22 * r22 * t2
    mx = fx * x / z + cx
    my = fy * y / z + cy
    a1 = fx / z
    b2 = fy / z
    c1 = -fx * x / (z * z)
    c2 = -fy * y / (z * z)
    cov00 = a1 * a1 * s00 + 2 * a1 * c1 * s02 + c1 * c1 * s22 + 1e-4
    cov01 = a1 * b2 * s01 + a1 * c2 * s02 + c1 * b2 * s12 + c1 * c2 * s22
    cov11 = b2 * b2 * s11 + 2 * b2 * c2 * s12 + c2 * c2 * s22 + 1e-4
    trace = cov00 + cov11
    det = cov00 * cov11 - cov01 * cov01
    lam = (trace + jnp.sqrt(jnp.maximum(trace * trace - 4 * det, 0.0))) * 0.5
    radius = 3.0 * jnp.sqrt(lam + 1e-6)
    ftw = jnp.float32(TW - 1)
    fth = jnp.float32(TH - 1)
    txmin = jnp.clip(jnp.floor((mx - radius) * (1.0 / TILE)), 0.0, ftw).astype(jnp.int32)
    txmax = jnp.clip(jnp.ceil((mx + radius) * (1.0 / TILE)), 0.0, ftw).astype(jnp.int32)
    tymin = jnp.clip(jnp.floor((my - radius) * (1.0 / TILE)), 0.0, fth).astype(jnp.int32)
    tymax = jnp.clip(jnp.ceil((my + radius) * (1.0 / TILE)), 0.0, fth).astype(jnp.int32)
    one32 = jnp.int32(1)
    zero_row = jnp.zeros((1, n), jnp.int32)
    lo = zero_row
    hi = zero_row
    for t in range(NT):
        ty_t = t // TW
        tx_t = t % TW
        cov = ((txmin <= tx_t) & (tx_t <= txmax)
               & (tymin <= ty_t) & (ty_t <= tymax))
        bit = jnp.left_shift(one32, t % 32)
        if t < 32:
            lo = lo | jnp.where(cov, bit, 0)
        else:
            hi = hi | jnp.where(cov, bit, 0)
    inv_det = 1.0 / (cov00 * cov11 - cov01 * cov01 + 1e-12)
    ci00 = cov11 * inv_det
    cs = -(cov01 + cov01) * inv_det
    ci11 = cov00 * inv_det
    nl = n // 8
    key = jnp.reshape(-z, (8, nl))
    si = lax.broadcasted_iota(jnp.int32, (8, nl), 0)
    li = lax.broadcasted_iota(jnp.int32, (8, nl), 1)
    pay = si * nl + li
    kk = 2
    while kk <= n:
        jj = kk // 2
        while jj >= 1:
            if jj < nl:
                high = (li & jj) != 0
                tk = jnp.where(high, _rollr(key, jj), _rolll(key, jj))
                tp = jnp.where(high, _rollr(pay, jj), _rolll(pay, jj))
            else:
                js = jj // nl
                high = (si & js) != 0
                tk = jnp.where(high, _rollrs(key, js), _rollls(key, js))
                tp = jnp.where(high, _rollrs(pay, js), _rollls(pay, js))
            if kk < nl:
                desc = (li & kk) != 0
            else:
                desc = (si & (kk // nl)) != 0
            lexgt = (key > tk) | ((key == tk) & (pay > tp))
            take = lexgt ^ high ^ desc
            key = jnp.where(take, tk, key)
            pay = jnp.where(take, tp, pay)
            jj //= 2
        kk *= 2
    order_ref[...] = jnp.reshape(pay, (1, n))
    table_ref[0:1, :] = mx
    table_ref[1:2, :] = my
    table_ref[2:3, :] = ci00
    table_ref[3:4, :] = cs
    table_ref[4:5, :] = ci11
    table_ref[5:6, :] = inp_ref[13:14, :]
    table_ref[6:9, :] = inp_ref[10:13, :]
    table_ref[9:10, :] = z
    table_ref[10:11, :] = jnp.ones((1, n), jnp.float32)
    table_ref[11:16, :] = jnp.zeros((5, n), jnp.float32)
    bounds_ref[0:1, :] = lo
    bounds_ref[1:2, :] = hi


def _make_sc_binner(n):
    nchunks = n // 16
    mesh = plsc.VectorSubcoreMesh(core_axis_name="c", subcore_axis_name="s")

    @functools.partial(
        pl.kernel, mesh=mesh,
        out_type=[jax.ShapeDtypeStruct((NT, 16, n), jnp.float32),
                  jax.ShapeDtypeStruct((_NW, 16), jnp.int32)],
        scratch_types=[pltpu.VMEM((16, n), jnp.float32),
                       pltpu.VMEM((2, n), jnp.int32),
                       pltpu.VMEM((1, n), jnp.int32),
                       pltpu.VMEM((16, n), jnp.float32),
                       pltpu.VMEM((16, n), jnp.float32),
                       pltpu.VMEM((16,), jnp.int32)],
        compiler_params=pltpu.CompilerParams(needs_layout_passes=False),
    )
    def binner(tablet_hbm, bounds_hbm, order_hbm, gath_out, cnt_out,
               table_v, bounds_v, order_v, rows_v, rows2_v, cnt_v):
        wid = lax.axis_index("s") * _NC + lax.axis_index("c")
        pltpu.sync_copy(tablet_hbm, table_v)
        pltpu.sync_copy(bounds_hbm, bounds_v)
        pltpu.sync_copy(order_hbm, order_v)
        iota = lax.broadcasted_iota(jnp.int32, (16,), 0)
        lane15 = jnp.full((16,), 15, jnp.int32)
        row_ids = [jnp.full((16,), r, jnp.int32) for r in range(11)]
        ones16 = jnp.ones((16,), jnp.float32)
        gdn = lax.GatherDimensionNumbers(offset_dims=(),
                                         collapsed_slice_dims=(0,),
                                         start_index_map=(0,))
        t0 = wid * _TPW
        wrow = jnp.zeros((16,), jnp.int32) + (t0 // 32)
        bit0 = t0 % 32
        bit1 = (t0 + 1) % 32
        base0 = jnp.zeros((16,), jnp.int32)
        base1 = jnp.zeros((16,), jnp.int32)
        for j in range(nchunks):
            ordc = order_v[0, pl.ds(j * 16, 16)]
            wc = plsc.load_gather(bounds_v, [wrow, ordc])
            m0 = (lax.shift_right_logical(wc, bit0) & 1) != 0
            m1 = (lax.shift_right_logical(wc, bit1) & 1) != 0
            cum0 = plsc.cumsum(jnp.where(m0, 1, 0))
            cum1 = plsc.cumsum(jnp.where(m1, 1, 0))
            pos0 = base0 + cum0 - 1
            pos1 = base1 + cum1 - 1
            for p in range(10):
                vals = plsc.load_gather(table_v, [row_ids[p], ordc])
                plsc.store_scatter(rows_v, [row_ids[p], pos0], vals, mask=m0)
                plsc.store_scatter(rows2_v, [row_ids[p], pos1], vals, mask=m1)
            plsc.store_scatter(rows_v, [row_ids[10], pos0], ones16, mask=m0)
            plsc.store_scatter(rows2_v, [row_ids[10], pos1], ones16, mask=m1)
            base0 = base0 + lax.gather(cum0, lane15[:, None], gdn, (1,),
                                       mode=lax.GatherScatterMode.PROMISE_IN_BOUNDS)
            base1 = base1 + lax.gather(cum1, lane15[:, None], gdn, (1,),
                                       mode=lax.GatherScatterMode.PROMISE_IN_BOUNDS)
        pltpu.sync_copy(rows_v, gath_out.at[t0])
        pltpu.sync_copy(rows2_v, gath_out.at[t0 + 1])
        counts_vec = jnp.where(iota == 0, base0, jnp.zeros((16,), jnp.int32))
        counts_vec = jnp.where(iota == 1, base1, counts_vec)
        cnt_v[...] = counts_vec
        pltpu.sync_copy(cnt_v, cnt_out.at[wid])

    return binner


_TB = 8


def _raster_kernel(cnt_ref, p_ref, o_ref):
    pid = pl.program_id(0)
    idx = lax.broadcasted_iota(jnp.int32, (P, 1), 0)
    iyf = (idx // TILE).astype(jnp.float32)
    ixf = (idx % TILE).astype(jnp.float32)
    lane = lax.broadcasted_iota(jnp.int32, (1, _G), 1)
    ur = lax.broadcasted_iota(jnp.int32, (_G, 1), 0)
    uc = lax.broadcasted_iota(jnp.int32, (1, _G), 1)
    ut = jnp.where(ur < uc, 1.0, 0.0)

    for tt in range(_TB):
        t = pid * _TB + tt
        tyf = (t // TW).astype(jnp.float32)
        txf = (t % TW).astype(jnp.float32)
        pyf = tyf * TILE + iyf
        pxf = txf * TILE + ixf
        cnt = cnt_ref[t]
        nch = (cnt + _G - 1) // _G

        def body(c, carry, tt=tt, cnt=cnt, pxf=pxf, pyf=pyf):
            acc, tcar = carry
            sl = pl.ds(c * _G, _G)
            inb = (c * _G + lane) < cnt
            mx = p_ref[tt, 0:1, sl]
            my = p_ref[tt, 1:2, sl]
            ci00 = p_ref[tt, 2:3, sl]
            cs = p_ref[tt, 3:4, sl]
            ci11 = p_ref[tt, 4:5, sl]
            op = p_ref[tt, 5:6, sl]
            dx = pxf - mx
            dy = pyf - my
            mahal = dx * dx * ci00 + dx * dy * cs + dy * dy * ci11
            w = jnp.exp(-0.5 * mahal) * op
            alpha = jnp.where((w > 0.01) & inb, w, 0.0)
            lg = jnp.log(1.0 - alpha)
            cume = lax.dot_general(lg, ut, (((1,), (0,)), ((), ())),
                                   preferred_element_type=jnp.float32)
            texcl = jnp.exp(cume) * tcar
            contrib = texcl * alpha
            p8 = jnp.where(inb, p_ref[tt, 6:14, sl], 0.0)
            acc = acc + lax.dot_general(p8, contrib,
                                        (((1,), (1,)), ((), ())),
                                        preferred_element_type=jnp.float32)
            tcar = tcar * jnp.exp(jnp.sum(lg, axis=1, keepdims=True))
            return acc, tcar

        acc, _ = lax.fori_loop(
            0, nch, body,
            (jnp.zeros((8, P), jnp.float32), jnp.ones((P, 1), jnp.float32)))
        o_ref[tt] = acc


def kernel(positions, scales, rotations, colors, opacities, K, background):
    n = positions.shape[0]
    inp = jnp.concatenate([positions.T, scales.T, rotations.T,
                           colors.T, opacities.T], axis=0).astype(jnp.float32)
    kvec = jnp.stack([K[0, 0], K[1, 1], K[0, 2], K[1, 2]]).astype(jnp.float32)
    table, bounds, order = pl.pallas_call(
        _prep_kernel,
        in_specs=[pl.BlockSpec(memory_space=pltpu.VMEM),
                  pl.BlockSpec(memory_space=pltpu.SMEM)],
        out_specs=[pl.BlockSpec(memory_space=pltpu.VMEM),
                   pl.BlockSpec(memory_space=pltpu.VMEM),
                   pl.BlockSpec(memory_space=pltpu.VMEM)],
        out_shape=[jax.ShapeDtypeStruct((16, n), jnp.float32),
                   jax.ShapeDtypeStruct((2, n), jnp.int32),
                   jax.ShapeDtypeStruct((1, n), jnp.int32)],
    )(inp, kvec)
    gathered, cnt32 = _make_sc_binner(n)(table, bounds, order)
    counts = cnt32[:, :_TPW].reshape(-1)
    grid_spec = pltpu.PrefetchScalarGridSpec(
        num_scalar_prefetch=1,
        grid=(NT // _TB,),
        in_specs=[pl.BlockSpec((_TB, 16, n), lambda t, s: (t, 0, 0))],
        out_specs=pl.BlockSpec((_TB, 8, P), lambda t, s: (t, 0, 0)),
    )
    out = pl.pallas_call(
        _raster_kernel,
        grid_spec=grid_spec,
        out_shape=jax.ShapeDtypeStruct((NT, 8, P), jnp.float32),
    )(counts, gathered)
    r = out.reshape(TH, TW, 8, TILE, TILE).transpose(2, 0, 3, 1, 4).reshape(8, H, W)
    img = jnp.clip(background[None, None, :] + r[0:3].transpose(1, 2, 0), 0.0, 1.0)
    return img, r[3], r[4]
